# baseline (speedup 1.0000x reference)
import jax
import jax.numpy as jnp
from jax import lax
from jax.experimental import pallas as pl
from jax.experimental.pallas import tpu as pltpu
from jax.experimental.pallas import tpu_sc as plsc

_DECAY = 0.1
_NW = 32
_MCHK = 2000
_CHK = 2048
_FB = 8
_D = 32
_N = 1000000
_NCH = _N // _CHK
_TAIL = _N - _NCH * _CHK


def _mesh():
    return plsc.VectorSubcoreMesh(core_axis_name="c", subcore_axis_name="s")


def _wid():
    return lax.axis_index("s") * 2 + lax.axis_index("c")


def _max_kernel(t_hbm, out_hbm, tbuf, mbuf, sem):
    w = _wid()
    nch = t_hbm.shape[0] // _MCHK
    k_w = (nch - w + _NW - 1) // _NW

    def chunk_body(k, acc):
        c = w + _NW * k
        off = pl.multiple_of(c * _MCHK, 8)
        pltpu.async_copy(t_hbm.at[pl.ds(off, _MCHK)], tbuf, sem).wait()

        def vbody(i, acc):
            return jnp.maximum(acc, tbuf[pl.ds(16 * i, 16)])

        return lax.fori_loop(0, _MCHK // 16, vbody, acc)

    acc = jnp.full((16,), -jnp.inf, dtype=jnp.float32)
    acc = lax.fori_loop(0, k_w, chunk_body, acc)
    mbuf[...] = acc
    pltpu.sync_copy(mbuf, out_hbm.at[pl.ds(pl.multiple_of(w * 16, 8), 16)])


def _main_kernel(maxes_hbm, t_hbm, x_hbm, p_hbm, o_hbm,
                 mbuf, tb0, tb1, xb0, xb1, pb0, pb1, ob0, ob1,
                 isem0, isem1, osem0, osem1):
    w = _wid()
    fb = w % 4
    f0 = pl.multiple_of(fb * _FB, 8)

    tb = (tb0, tb1)
    xb = (xb0, xb1)
    pb = (pb0, pb1)
    ob = (ob0, ob1)
    isem = (isem0, isem1)
    osem = (osem0, osem1)

    pltpu.sync_copy(maxes_hbm, mbuf)

    def mbody(i, acc):
        return jnp.maximum(acc, mbuf[pl.ds(16 * i, 16)])

    macc = lax.fori_loop(0, _NW, mbody, jnp.full((16,), -jnp.inf, jnp.float32))
    smax = macc[0]
    for i in range(1, 16):
        smax = jnp.maximum(smax, macc[i])
    si = smax.astype(jnp.int32)
    steps_i = jnp.where(si.astype(jnp.float32) > smax, si - 1, si)

    k_w = 61

    def chunk_of(k):
        return w // 4 + 8 * k

    def start_in(b, k):
        s0 = pl.multiple_of(chunk_of(k) * _CHK, _CHK)
        pltpu.async_copy(t_hbm.at[pl.ds(s0, _CHK)], tb[b], isem[b])
        pltpu.async_copy(x_hbm.at[pl.ds(f0, _FB), pl.ds(s0, _CHK)], xb[b], isem[b])
        pltpu.async_copy(p_hbm.at[pl.ds(f0, _FB), pl.ds(s0, _CHK)], pb[b], isem[b])

    def wait_in(b):
        pltpu.make_async_copy(t_hbm.at[pl.ds(0, _CHK)], tb[b], isem[b]).wait()
        pltpu.make_async_copy(x_hbm.at[pl.ds(0, _FB), pl.ds(0, _CHK)], xb[b], isem[b]).wait()
        pltpu.make_async_copy(p_hbm.at[pl.ds(0, _FB), pl.ds(0, _CHK)], pb[b], isem[b]).wait()

    def start_out(b, k):
        s0 = pl.multiple_of(chunk_of(k) * _CHK, _CHK)
        pltpu.async_copy(ob[b], o_hbm.at[pl.ds(f0, _FB), pl.ds(s0, _CHK)], osem[b])

    def wait_out(b):
        pltpu.make_async_copy(ob[b], o_hbm.at[pl.ds(0, _FB), pl.ds(0, _CHK)], osem[b]).wait()

    def compute(tbr, xbr, pbr, obr, ngroups):
        def gbody(g, _):
            t16 = tbr[pl.ds(16 * g, 16)]
            ti = t16.astype(jnp.int32)
            n = jnp.where(t16 > ti.astype(jnp.float32), ti + 1, ti)
            n = jnp.minimum(n, steps_i)
            m0 = (n & 1) > 0
            m1 = (n & 2) > 0
            m2 = (n & 4) > 0
            one = jnp.full((16,), 1.0, dtype=jnp.float32)
            for d in range(_FB):
                x = xbr[d, pl.ds(16 * g, 16)]
                p = pbr[d, pl.ds(16 * g, 16)]
                f = 1.0 - _DECAY * p
                f2 = f * f
                f4 = f2 * f2
                y = x * jnp.where(m0, f, one)
                y = y * jnp.where(m1, f2, one)
                y = y * jnp.where(m2, f4, one)
                obr[d, pl.ds(16 * g, 16)] = y
            return 0

        lax.fori_loop(0, ngroups, gbody, 0)

    start_in(0, 0)

    def outer(k0, _):
        for b in range(2):
            k = 2 * k0 + b

            @pl.when(k < k_w)
            def _iter():
                @pl.when(k + 1 < k_w)
                def _prefetch():
                    start_in((b + 1) % 2, k + 1)

                wait_in(b)

                @pl.when(k >= 2)
                def _reuse():
                    wait_out(b)

                compute(tb[b], xb[b], pb[b], ob[b], _CHK // 16)
                start_out(b, k)

        return 0

    lax.fori_loop(0, (k_w + 1) // 2, outer, 0)

    for b in range(2):
        @pl.when(k_w >= b + 1)
        def _drain():
            wait_out(b)


def _tc_tail_body(steps_ref, big_ref, t_ref, x_ref, p_ref, o_ref):
    steps = steps_ref[0]
    n = jnp.minimum(jnp.ceil(t_ref[...]).astype(jnp.int32), steps)
    b0 = ((n & 1) > 0)[None, :]
    b1 = ((n & 2) > 0)[None, :]
    b2 = ((n & 4) > 0)[None, :]
    f = 1.0 - _DECAY * p_ref[...]
    f2 = f * f
    f4 = f2 * f2
    one = jnp.float32(1.0)
    m = jnp.where(b0, f, one)
    m = m * jnp.where(b1, f2, one)
    m = m * jnp.where(b2, f4, one)
    o_ref[...] = x_ref[...] * m


def kernel(X, T, params):
    N, D = X.shape
    xt = X.T
    pt = params.T

    maxes = pl.kernel(
        _max_kernel,
        mesh=_mesh(),
        out_type=jax.ShapeDtypeStruct((_NW * 16,), jnp.float32),
        scratch_types=[
            pltpu.VMEM((_MCHK,), jnp.float32),
            pltpu.VMEM((16,), jnp.float32),
            pltpu.SemaphoreType.DMA,
        ],
    )(T)

    out_t = pl.kernel(
        _main_kernel,
        mesh=_mesh(),
        out_type=jax.ShapeDtypeStruct((D, N), jnp.float32),
        scratch_types=[
            pltpu.VMEM((_NW * 16,), jnp.float32),
            pltpu.VMEM((_CHK,), jnp.float32),
            pltpu.VMEM((_CHK,), jnp.float32),
            pltpu.VMEM((_FB, _CHK), jnp.float32),
            pltpu.VMEM((_FB, _CHK), jnp.float32),
            pltpu.VMEM((_FB, _CHK), jnp.float32),
            pltpu.VMEM((_FB, _CHK), jnp.float32),
            pltpu.VMEM((_FB, _CHK), jnp.float32),
            pltpu.VMEM((_FB, _CHK), jnp.float32),
            pltpu.SemaphoreType.DMA,
            pltpu.SemaphoreType.DMA,
            pltpu.SemaphoreType.DMA,
            pltpu.SemaphoreType.DMA,
        ],
    )(maxes, T, xt, pt)

    steps_arr = jnp.floor(jnp.max(maxes)).astype(jnp.int32).reshape(1)
    bt = 512
    base = (_NCH * _CHK) // bt
    out_t = pl.pallas_call(
        _tc_tail_body,
        grid=(2,),
        in_specs=[
            pl.BlockSpec(memory_space=pltpu.SMEM),
            pl.BlockSpec((D, bt), lambda i: (0, base + i)),
            pl.BlockSpec((bt,), lambda i: (base + i,)),
            pl.BlockSpec((D, bt), lambda i: (0, base + i)),
            pl.BlockSpec((D, bt), lambda i: (0, base + i)),
        ],
        out_specs=pl.BlockSpec((D, bt), lambda i: (0, base + i)),
        out_shape=jax.ShapeDtypeStruct((D, N), jnp.float32),
        input_output_aliases={1: 0},
    )(steps_arr, out_t, T, xt, pt)

    return out_t.T


# speedup vs baseline: 1.0188x; 1.0188x over previous
import jax
import jax.numpy as jnp
from jax import lax
from jax.experimental import pallas as pl
from jax.experimental.pallas import tpu as pltpu
from jax.experimental.pallas import tpu_sc as plsc

_DECAY = 0.1
_NW = 32
_MCHK = 20000
_CHK = 2048
_FB = 8
_D = 32
_N = 1000000
_NCH = _N // _CHK
_TAIL = _N - _NCH * _CHK


def _mesh():
    return plsc.VectorSubcoreMesh(core_axis_name="c", subcore_axis_name="s")


def _wid():
    return lax.axis_index("s") * 2 + lax.axis_index("c")


def _max_kernel(t_hbm, out_hbm, tbuf, mbuf, sem):
    w = _wid()
    nch = t_hbm.shape[0] // _MCHK
    k_w = (nch - w + _NW - 1) // _NW

    def chunk_body(k, acc):
        c = w + _NW * k
        off = pl.multiple_of(c * _MCHK, 8)
        pltpu.async_copy(t_hbm.at[pl.ds(off, _MCHK)], tbuf, sem).wait()

        def vbody(i, acc):
            return jnp.maximum(acc, tbuf[pl.ds(16 * i, 16)])

        return lax.fori_loop(0, _MCHK // 16, vbody, acc)

    acc = jnp.full((16,), -jnp.inf, dtype=jnp.float32)
    acc = lax.fori_loop(0, k_w, chunk_body, acc)
    mbuf[...] = acc
    pltpu.sync_copy(mbuf, out_hbm.at[pl.ds(pl.multiple_of(w * 16, 8), 16)])


def _main_kernel(maxes_hbm, t_hbm, x_hbm, p_hbm, o_hbm,
                 mbuf, tb0, tb1, xb0, xb1, pb0, pb1, ob0, ob1,
                 isem0, isem1, osem0, osem1):
    w = _wid()
    fb = w % 4
    f0 = pl.multiple_of(fb * _FB, 8)

    tb = (tb0, tb1)
    xb = (xb0, xb1)
    pb = (pb0, pb1)
    ob = (ob0, ob1)
    isem = (isem0, isem1)
    osem = (osem0, osem1)

    pltpu.sync_copy(maxes_hbm, mbuf)

    def mbody(i, acc):
        return jnp.maximum(acc, mbuf[pl.ds(16 * i, 16)])

    macc = lax.fori_loop(0, _NW, mbody, jnp.full((16,), -jnp.inf, jnp.float32))
    smax = macc[0]
    for i in range(1, 16):
        smax = jnp.maximum(smax, macc[i])
    si = smax.astype(jnp.int32)
    steps_i = jnp.where(si.astype(jnp.float32) > smax, si - 1, si)

    k_w = 61

    def chunk_of(k):
        return w // 4 + 8 * k

    def start_in(b, k):
        s0 = pl.multiple_of(chunk_of(k) * _CHK, _CHK)
        pltpu.async_copy(t_hbm.at[pl.ds(s0, _CHK)], tb[b], isem[b])
        pltpu.async_copy(x_hbm.at[pl.ds(f0, _FB), pl.ds(s0, _CHK)], xb[b], isem[b])
        pltpu.async_copy(p_hbm.at[pl.ds(f0, _FB), pl.ds(s0, _CHK)], pb[b], isem[b])

    def wait_in(b):
        pltpu.make_async_copy(t_hbm.at[pl.ds(0, _CHK)], tb[b], isem[b]).wait()
        pltpu.make_async_copy(x_hbm.at[pl.ds(0, _FB), pl.ds(0, _CHK)], xb[b], isem[b]).wait()
        pltpu.make_async_copy(p_hbm.at[pl.ds(0, _FB), pl.ds(0, _CHK)], pb[b], isem[b]).wait()

    def start_out(b, k):
        s0 = pl.multiple_of(chunk_of(k) * _CHK, _CHK)
        pltpu.async_copy(ob[b], o_hbm.at[pl.ds(f0, _FB), pl.ds(s0, _CHK)], osem[b])

    def wait_out(b):
        pltpu.make_async_copy(ob[b], o_hbm.at[pl.ds(0, _FB), pl.ds(0, _CHK)], osem[b]).wait()

    def compute(tbr, xbr, pbr, obr, ngroups):
        def gbody(g, _):
            t16 = tbr[pl.ds(16 * g, 16)]
            ti = t16.astype(jnp.int32)
            n = jnp.where(t16 > ti.astype(jnp.float32), ti + 1, ti)
            n = jnp.minimum(n, steps_i)
            m0 = (n & 1) > 0
            m1 = (n & 2) > 0
            m2 = (n & 4) > 0
            one = jnp.full((16,), 1.0, dtype=jnp.float32)
            for d in range(_FB):
                x = xbr[d, pl.ds(16 * g, 16)]
                p = pbr[d, pl.ds(16 * g, 16)]
                f = 1.0 - _DECAY * p
                f2 = f * f
                f4 = f2 * f2
                y = x * jnp.where(m0, f, one)
                y = y * jnp.where(m1, f2, one)
                y = y * jnp.where(m2, f4, one)
                obr[d, pl.ds(16 * g, 16)] = y
            return 0

        lax.fori_loop(0, ngroups, gbody, 0)

    start_in(0, 0)

    def outer(k0, _):
        for b in range(2):
            k = 2 * k0 + b

            @pl.when(k < k_w)
            def _iter():
                @pl.when(k + 1 < k_w)
                def _prefetch():
                    start_in((b + 1) % 2, k + 1)

                wait_in(b)

                @pl.when(k >= 2)
                def _reuse():
                    wait_out(b)

                compute(tb[b], xb[b], pb[b], ob[b], _CHK // 16)
                start_out(b, k)

        return 0

    lax.fori_loop(0, (k_w + 1) // 2, outer, 0)

    for b in range(2):
        @pl.when(k_w >= b + 1)
        def _drain():
            wait_out(b)


def _tc_tail_body(steps_ref, big_ref, t_ref, x_ref, p_ref, o_ref):
    steps = steps_ref[0]
    n = jnp.minimum(jnp.ceil(t_ref[...]).astype(jnp.int32), steps)
    b0 = ((n & 1) > 0)[None, :]
    b1 = ((n & 2) > 0)[None, :]
    b2 = ((n & 4) > 0)[None, :]
    f = 1.0 - _DECAY * p_ref[...]
    f2 = f * f
    f4 = f2 * f2
    one = jnp.float32(1.0)
    m = jnp.where(b0, f, one)
    m = m * jnp.where(b1, f2, one)
    m = m * jnp.where(b2, f4, one)
    o_ref[...] = x_ref[...] * m


def kernel(X, T, params):
    N, D = X.shape
    xt = X.T
    pt = params.T

    maxes = pl.kernel(
        _max_kernel,
        mesh=_mesh(),
        out_type=jax.ShapeDtypeStruct((_NW * 16,), jnp.float32),
        scratch_types=[
            pltpu.VMEM((_MCHK,), jnp.float32),
            pltpu.VMEM((16,), jnp.float32),
            pltpu.SemaphoreType.DMA,
        ],
    )(T)

    out_t = pl.kernel(
        _main_kernel,
        mesh=_mesh(),
        out_type=jax.ShapeDtypeStruct((D, N), jnp.float32),
        scratch_types=[
            pltpu.VMEM((_NW * 16,), jnp.float32),
            pltpu.VMEM((_CHK,), jnp.float32),
            pltpu.VMEM((_CHK,), jnp.float32),
            pltpu.VMEM((_FB, _CHK), jnp.float32),
            pltpu.VMEM((_FB, _CHK), jnp.float32),
            pltpu.VMEM((_FB, _CHK), jnp.float32),
            pltpu.VMEM((_FB, _CHK), jnp.float32),
            pltpu.VMEM((_FB, _CHK), jnp.float32),
            pltpu.VMEM((_FB, _CHK), jnp.float32),
            pltpu.SemaphoreType.DMA,
            pltpu.SemaphoreType.DMA,
            pltpu.SemaphoreType.DMA,
            pltpu.SemaphoreType.DMA,
        ],
    )(maxes, T, xt, pt)

    steps_arr = jnp.floor(jnp.max(maxes)).astype(jnp.int32).reshape(1)
    bt = 512
    base = (_NCH * _CHK) // bt
    out_t = pl.pallas_call(
        _tc_tail_body,
        grid=(2,),
        in_specs=[
            pl.BlockSpec(memory_space=pltpu.SMEM),
            pl.BlockSpec((D, bt), lambda i: (0, base + i)),
            pl.BlockSpec((bt,), lambda i: (base + i,)),
            pl.BlockSpec((D, bt), lambda i: (0, base + i)),
            pl.BlockSpec((D, bt), lambda i: (0, base + i)),
        ],
        out_specs=pl.BlockSpec((D, bt), lambda i: (0, base + i)),
        out_shape=jax.ShapeDtypeStruct((D, N), jnp.float32),
        input_output_aliases={1: 0},
    )(steps_arr, out_t, T, xt, pt)

    return out_t.T


# speedup vs baseline: 1.5621x; 1.5333x over previous
import jax
import jax.numpy as jnp
from jax import lax
from jax.experimental import pallas as pl
from jax.experimental.pallas import tpu as pltpu
from jax.experimental.pallas import tpu_sc as plsc

_DECAY = 0.1
_NW = 32
_MCHK = 20000
_CHK = 2048
_FB = 8
_D = 32
_N = 1000000
_NCH = _N // _CHK
_TAIL = _N - _NCH * _CHK


def _mesh():
    return plsc.VectorSubcoreMesh(core_axis_name="c", subcore_axis_name="s")


def _wid():
    return lax.axis_index("s") * 2 + lax.axis_index("c")


def _max_kernel(t_hbm, out_hbm, tbuf, mbuf, sem):
    w = _wid()
    nch = t_hbm.shape[0] // _MCHK
    k_w = (nch - w + _NW - 1) // _NW

    def chunk_body(k, acc):
        c = w + _NW * k
        off = pl.multiple_of(c * _MCHK, 8)
        pltpu.async_copy(t_hbm.at[pl.ds(off, _MCHK)], tbuf, sem).wait()

        def vbody(i, acc):
            return jnp.maximum(acc, tbuf[pl.ds(16 * i, 16)])

        return lax.fori_loop(0, _MCHK // 16, vbody, acc)

    acc = jnp.full((16,), -jnp.inf, dtype=jnp.float32)
    acc = lax.fori_loop(0, k_w, chunk_body, acc)
    mbuf[...] = acc
    pltpu.sync_copy(mbuf, out_hbm.at[pl.ds(pl.multiple_of(w * 16, 8), 16)])


def _main_kernel(maxes_hbm, t_hbm, x_hbm, p_hbm, o_hbm,
                 mbuf, tb0, tb1, xb0, xb1, pb0, pb1, ob0, ob1,
                 isem0, isem1, osem0, osem1):
    w = _wid()
    fb = w % 4
    f0 = pl.multiple_of(fb * _FB, 8)

    tb = (tb0, tb1)
    xb = (xb0, xb1)
    pb = (pb0, pb1)
    ob = (ob0, ob1)
    isem = (isem0, isem1)
    osem = (osem0, osem1)

    pltpu.sync_copy(maxes_hbm, mbuf)

    def mbody(i, acc):
        return jnp.maximum(acc, mbuf[pl.ds(16 * i, 16)])

    macc = lax.fori_loop(0, _NW, mbody, jnp.full((16,), -jnp.inf, jnp.float32))
    smax = macc[0]
    for i in range(1, 16):
        smax = jnp.maximum(smax, macc[i])
    si = smax.astype(jnp.int32)
    steps_i = jnp.where(si.astype(jnp.float32) > smax, si - 1, si)

    k_w = 61

    def chunk_of(k):
        return w // 4 + 8 * k

    def start_in(b, k):
        s0 = pl.multiple_of(chunk_of(k) * _CHK, _CHK)
        pltpu.async_copy(t_hbm.at[pl.ds(s0, _CHK)], tb[b], isem[b])
        pltpu.async_copy(x_hbm.at[pl.ds(f0, _FB), pl.ds(s0, _CHK)], xb[b], isem[b])
        pltpu.async_copy(p_hbm.at[pl.ds(f0, _FB), pl.ds(s0, _CHK)], pb[b], isem[b])

    def wait_in(b):
        pltpu.make_async_copy(t_hbm.at[pl.ds(0, _CHK)], tb[b], isem[b]).wait()
        pltpu.make_async_copy(x_hbm.at[pl.ds(0, _FB), pl.ds(0, _CHK)], xb[b], isem[b]).wait()
        pltpu.make_async_copy(p_hbm.at[pl.ds(0, _FB), pl.ds(0, _CHK)], pb[b], isem[b]).wait()

    def start_out(b, k):
        s0 = pl.multiple_of(chunk_of(k) * _CHK, _CHK)
        pltpu.async_copy(ob[b], o_hbm.at[pl.ds(f0, _FB), pl.ds(s0, _CHK)], osem[b])

    def wait_out(b):
        pltpu.make_async_copy(ob[b], o_hbm.at[pl.ds(0, _FB), pl.ds(0, _CHK)], osem[b]).wait()

    def compute(tbr, xbr, pbr, obr, ngroups):
        @plsc.parallel_loop(0, ngroups * 16, step=16, unroll=2)
        def gbody(i):
            t16 = tbr[pl.ds(i, 16)]
            ti = t16.astype(jnp.int32)
            n = jnp.where(t16 > ti.astype(jnp.float32), ti + 1, ti)
            n = jnp.minimum(n, steps_i)
            m0 = (n & 1) > 0
            m1 = (n & 2) > 0
            m2 = (n & 4) > 0
            one = jnp.full((16,), 1.0, dtype=jnp.float32)
            for d in range(_FB):
                x = xbr[d, pl.ds(i, 16)]
                p = pbr[d, pl.ds(i, 16)]
                f = 1.0 - _DECAY * p
                f2 = f * f
                f4 = f2 * f2
                y = x * jnp.where(m0, f, one)
                y = y * jnp.where(m1, f2, one)
                y = y * jnp.where(m2, f4, one)
                obr[d, pl.ds(i, 16)] = y

    start_in(0, 0)

    def outer(k0, _):
        for b in range(2):
            k = 2 * k0 + b

            @pl.when(k < k_w)
            def _iter():
                @pl.when(k + 1 < k_w)
                def _prefetch():
                    start_in((b + 1) % 2, k + 1)

                wait_in(b)

                @pl.when(k >= 2)
                def _reuse():
                    wait_out(b)

                compute(tb[b], xb[b], pb[b], ob[b], _CHK // 16)
                start_out(b, k)

        return 0

    lax.fori_loop(0, (k_w + 1) // 2, outer, 0)

    for b in range(2):
        @pl.when(k_w >= b + 1)
        def _drain():
            wait_out(b)


def _tc_tail_body(steps_ref, big_ref, t_ref, x_ref, p_ref, o_ref):
    steps = steps_ref[0]
    n = jnp.minimum(jnp.ceil(t_ref[...]).astype(jnp.int32), steps)
    b0 = ((n & 1) > 0)[None, :]
    b1 = ((n & 2) > 0)[None, :]
    b2 = ((n & 4) > 0)[None, :]
    f = 1.0 - _DECAY * p_ref[...]
    f2 = f * f
    f4 = f2 * f2
    one = jnp.float32(1.0)
    m = jnp.where(b0, f, one)
    m = m * jnp.where(b1, f2, one)
    m = m * jnp.where(b2, f4, one)
    o_ref[...] = x_ref[...] * m


def kernel(X, T, params):
    N, D = X.shape
    xt = X.T
    pt = params.T

    maxes = pl.kernel(
        _max_kernel,
        mesh=_mesh(),
        out_type=jax.ShapeDtypeStruct((_NW * 16,), jnp.float32),
        scratch_types=[
            pltpu.VMEM((_MCHK,), jnp.float32),
            pltpu.VMEM((16,), jnp.float32),
            pltpu.SemaphoreType.DMA,
        ],
    )(T)

    out_t = pl.kernel(
        _main_kernel,
        mesh=_mesh(),
        out_type=jax.ShapeDtypeStruct((D, N), jnp.float32),
        scratch_types=[
            pltpu.VMEM((_NW * 16,), jnp.float32),
            pltpu.VMEM((_CHK,), jnp.float32),
            pltpu.VMEM((_CHK,), jnp.float32),
            pltpu.VMEM((_FB, _CHK), jnp.float32),
            pltpu.VMEM((_FB, _CHK), jnp.float32),
            pltpu.VMEM((_FB, _CHK), jnp.float32),
            pltpu.VMEM((_FB, _CHK), jnp.float32),
            pltpu.VMEM((_FB, _CHK), jnp.float32),
            pltpu.VMEM((_FB, _CHK), jnp.float32),
            pltpu.SemaphoreType.DMA,
            pltpu.SemaphoreType.DMA,
            pltpu.SemaphoreType.DMA,
            pltpu.SemaphoreType.DMA,
        ],
    )(maxes, T, xt, pt)

    steps_arr = jnp.floor(jnp.max(maxes)).astype(jnp.int32).reshape(1)
    bt = 512
    base = (_NCH * _CHK) // bt
    out_t = pl.pallas_call(
        _tc_tail_body,
        grid=(2,),
        in_specs=[
            pl.BlockSpec(memory_space=pltpu.SMEM),
            pl.BlockSpec((D, bt), lambda i: (0, base + i)),
            pl.BlockSpec((bt,), lambda i: (base + i,)),
            pl.BlockSpec((D, bt), lambda i: (0, base + i)),
            pl.BlockSpec((D, bt), lambda i: (0, base + i)),
        ],
        out_specs=pl.BlockSpec((D, bt), lambda i: (0, base + i)),
        out_shape=jax.ShapeDtypeStruct((D, N), jnp.float32),
        input_output_aliases={1: 0},
    )(steps_arr, out_t, T, xt, pt)

    return out_t.T
